# baseline (speedup 1.0000x reference)
import functools

import jax
import jax.numpy as jnp
from jax import lax
from jax.experimental import pallas as pl
from jax.experimental.pallas import tpu as pltpu
from jax.experimental.pallas import tpu_sc as plsc

B = 4096
L = 50
DIM = 64
N = B * L

NC = 2
NS = 16
NW = NC * NS
B_PER_W = B // NW
CHUNK_B = 16
NCHUNK = B_PER_W // CHUNK_B
NBUF = 2


def _sc_gather(idx_hbm, table_hbm, out_hbm, idx_v, rows_v, gsem, osem):
  wid = lax.axis_index("s") * NC + lax.axis_index("c")
  b0 = wid * B_PER_W
  pltpu.sync_copy(idx_hbm.at[pl.ds(b0, B_PER_W)], idx_v)

  def gather_start(g, buf):
    for j in range(CHUNK_B):
      pltpu.async_copy(
          table_hbm.at[idx_v.at[g * CHUNK_B + j]], rows_v.at[buf, j], gsem
      )

  def gather_wait(buf):
    for j in range(CHUNK_B):
      pltpu.make_async_copy(
          table_hbm.at[idx_v.at[0]], rows_v.at[buf, j], gsem
      ).wait()

  def out_start(g, buf):
    pltpu.async_copy(
        rows_v.at[buf], out_hbm.at[pl.ds(b0 + g * CHUNK_B, CHUNK_B)], osem
    )

  def out_wait(buf):
    pltpu.make_async_copy(
        rows_v.at[buf], out_hbm.at[pl.ds(b0, CHUNK_B)], osem
    ).wait()

  gather_start(0, 0)
  for g in range(NCHUNK):
    buf = g % NBUF
    nbuf = (g + 1) % NBUF
    gather_wait(buf)
    if g + 1 < NCHUNK:
      if g >= 1:
        out_wait(nbuf)
      gather_start(g + 1, nbuf)
    out_start(g, buf)
  out_wait(0)
  out_wait(1)


@jax.jit
def _embedding(idx2d, table):
  mesh = plsc.VectorSubcoreMesh(core_axis_name="c", subcore_axis_name="s")
  f = pl.kernel(
      _sc_gather,
      out_type=jax.ShapeDtypeStruct((B, L, DIM), jnp.float32),
      mesh=mesh,
      scratch_types=[
          pltpu.VMEM((B_PER_W, L), jnp.int32),
          pltpu.VMEM((NBUF, CHUNK_B, L, DIM), jnp.float32),
          pltpu.SemaphoreType.DMA,
          pltpu.SemaphoreType.DMA,
      ],
      compiler_params=pltpu.CompilerParams(use_tc_tiling_on_sc=False),
  )
  return f(idx2d, table)


def kernel(input, table):
  return _embedding(input.astype(jnp.int32), table)
